# baseline (speedup 1.0000x reference)
import functools

import jax
import jax.numpy as jnp
from jax import lax
from jax.experimental import pallas as pl
from jax.experimental.pallas import tpu as pltpu
from jax.experimental.pallas import tpu_sc as plsc

_NUM_PERSON = 1000000
_NUM_ITEM = 1000
_LATENT_DIM = 64
_BATCH = 16384


def _sc_gather(table, idx):
    info = plsc.get_sparse_core_info()
    nc, ns = info.num_cores, info.num_subcores
    nw = nc * ns
    b_per_w = _BATCH // nw

    mesh = plsc.VectorSubcoreMesh(core_axis_name="c", subcore_axis_name="s")

    @functools.partial(
        pl.kernel,
        mesh=mesh,
        out_type=jax.ShapeDtypeStruct((_BATCH, _LATENT_DIM), jnp.float32),
        compiler_params=pltpu.CompilerParams(use_tc_tiling_on_sc=False),
        scratch_types=[
            pltpu.VMEM((b_per_w,), jnp.int32),
            pltpu.VMEM((b_per_w, _LATENT_DIM), jnp.float32),
            pltpu.SemaphoreType.DMA,
        ],
    )
    def gather_kernel(table_hbm, idx_hbm, out_hbm, idx_v, rows_v, sem):
        wid = lax.axis_index("s") * nc + lax.axis_index("c")
        base = wid * b_per_w
        pltpu.sync_copy(idx_hbm.at[pl.ds(base, b_per_w)], idx_v)
        pltpu.async_copy(table_hbm.at[idx_v], rows_v, sem).wait()
        pltpu.sync_copy(rows_v, out_hbm.at[pl.ds(base, b_per_w)])

    return gather_kernel(table, idx)


def _tc_decode(gathered, diff):
    bb = 1024
    grid = (_BATCH // bb,)

    def body(g_ref, d_ref, o_ref):
        s = jnp.sum(g_ref[...], axis=1, keepdims=True)
        o_ref[...] = jax.nn.sigmoid(s + d_ref[...])

    return pl.pallas_call(
        body,
        grid=grid,
        in_specs=[
            pl.BlockSpec((bb, _LATENT_DIM), lambda i: (i, 0)),
            pl.BlockSpec((1, _NUM_ITEM), lambda i: (0, 0)),
        ],
        out_specs=pl.BlockSpec((bb, _NUM_ITEM), lambda i: (i, 0)),
        out_shape=jax.ShapeDtypeStruct((_BATCH, _NUM_ITEM), jnp.float32),
    )(gathered, diff)


def kernel(index, response, mask, ability_table, item_table):
    idx = index.astype(jnp.int32)
    gathered = _sc_gather(ability_table, idx)
    diff = item_table.reshape(1, _NUM_ITEM)
    out = _tc_decode(gathered, diff)
    return out[..., None]
